# Initial kernel scaffold; baseline (speedup 1.0000x reference)
import jax
import jax.numpy as jnp
from jax.experimental import pallas as pl


def kernel(input_, weight):
    raise NotImplementedError("write your pallas kernel here")


# speedup vs baseline: 2.1656x; 2.1656x over previous
import functools

import jax
import jax.numpy as jnp
from jax import lax
from jax.experimental import pallas as pl
from jax.experimental.pallas import tpu as pltpu
from jax.experimental.pallas import tpu_sc as plsc

_D = 64
_H = 50
_L = 16
_NC = 2
_NS = 16
_NW = _NC * _NS
_BAGS_PER_CHUNK = 2
_IDX_PER_CHUNK = _BAGS_PER_CHUNK * _H


@functools.lru_cache(maxsize=None)
def _make_kernel(B, V):
    bags_per_w = B // _NW
    chunks_per_w = bags_per_w // _BAGS_PER_CHUNK
    mesh = plsc.VectorSubcoreMesh(core_axis_name="c", subcore_axis_name="s")

    @functools.partial(
        pl.kernel,
        mesh=mesh,
        out_type=jax.ShapeDtypeStruct((B, _D), jnp.float32),
        scratch_types=[
            pltpu.VMEM((chunks_per_w, _IDX_PER_CHUNK), jnp.int32),
            pltpu.VMEM((_IDX_PER_CHUNK, _D), jnp.float32),
            pltpu.VMEM((bags_per_w, _D), jnp.float32),
            pltpu.SemaphoreType.DMA,
        ],
        compiler_params=pltpu.CompilerParams(use_tc_tiling_on_sc=False),
    )
    def k(idx_hbm, table_hbm, out_hbm, idx_v, rows_v, out_v, sem):
        wid = lax.axis_index("s") * _NC + lax.axis_index("c")
        pltpu.sync_copy(idx_hbm.at[pl.ds(wid * chunks_per_w, chunks_per_w)],
                        idx_v)

        inv = jnp.float32(1.0 / _H)

        def chunk_body(c, carry):
            pltpu.async_copy(table_hbm.at[idx_v.at[c]], rows_v, sem).wait()
            for bag in range(_BAGS_PER_CHUNK):
                base = bag * _H
                accs = [rows_v[base, pl.ds(kk * _L, _L)]
                        for kk in range(_D // _L)]
                for r in range(1, _H):
                    for kk in range(_D // _L):
                        accs[kk] = accs[kk] + rows_v[base + r,
                                                     pl.ds(kk * _L, _L)]
                obag = c * _BAGS_PER_CHUNK + bag
                for kk in range(_D // _L):
                    out_v[obag, pl.ds(kk * _L, _L)] = accs[kk] * inv
            return carry

        lax.fori_loop(0, chunks_per_w, chunk_body, 0)
        pltpu.sync_copy(out_v, out_hbm.at[pl.ds(wid * bags_per_w, bags_per_w)])

    return k


def kernel(input_, weight):
    B, H = input_.shape
    V = weight.shape[0]
    idx2 = input_.reshape(B // _BAGS_PER_CHUNK, _IDX_PER_CHUNK)
    return _make_kernel(B, V)(idx2, weight)


# speedup vs baseline: 2.4443x; 1.1287x over previous
import functools

import jax
import jax.numpy as jnp
from jax import lax
from jax.experimental import pallas as pl
from jax.experimental.pallas import tpu as pltpu
from jax.experimental.pallas import tpu_sc as plsc

_D = 64
_H = 50
_L = 16
_NC = 2
_NS = 16
_NW = _NC * _NS
_BAGS_PER_CHUNK = 2
_IDX_PER_CHUNK = _BAGS_PER_CHUNK * _H
_NBUF = 4


@functools.lru_cache(maxsize=None)
def _make_kernel(B, V):
    bags_per_w = B // _NW
    chunks_per_w = bags_per_w // _BAGS_PER_CHUNK
    n_groups = chunks_per_w // _NBUF
    mesh = plsc.VectorSubcoreMesh(core_axis_name="c", subcore_axis_name="s")

    @functools.partial(
        pl.kernel,
        mesh=mesh,
        out_type=jax.ShapeDtypeStruct((B, _D), jnp.float32),
        scratch_types=[
            pltpu.VMEM((chunks_per_w, _IDX_PER_CHUNK), jnp.int32),
            [pltpu.VMEM((_IDX_PER_CHUNK, _D), jnp.float32)
             for _ in range(_NBUF)],
            pltpu.VMEM((bags_per_w, _D), jnp.float32),
            [pltpu.SemaphoreType.DMA for _ in range(_NBUF)],
        ],
        compiler_params=pltpu.CompilerParams(use_tc_tiling_on_sc=False),
    )
    def k(idx_hbm, table_hbm, out_hbm, idx_v, bufs, out_v, sems):
        wid = lax.axis_index("s") * _NC + lax.axis_index("c")
        pltpu.sync_copy(idx_hbm.at[pl.ds(wid * chunks_per_w, chunks_per_w)],
                        idx_v)

        inv = jnp.float32(1.0 / _H)

        def start(c, b):
            pltpu.async_copy(table_hbm.at[idx_v.at[c]], bufs[b], sems[b])

        def wait(c, b):
            pltpu.make_async_copy(table_hbm.at[idx_v.at[c]], bufs[b],
                                  sems[b]).wait()

        def reduce_chunk(c, b):
            rows_v = bufs[b]
            for bag in range(_BAGS_PER_CHUNK):
                base = bag * _H
                accs = [rows_v[base, pl.ds(kk * _L, _L)]
                        for kk in range(_D // _L)]
                for r in range(1, _H):
                    for kk in range(_D // _L):
                        accs[kk] = accs[kk] + rows_v[base + r,
                                                     pl.ds(kk * _L, _L)]
                obag = c * _BAGS_PER_CHUNK + bag
                for kk in range(_D // _L):
                    out_v[obag, pl.ds(kk * _L, _L)] = accs[kk] * inv

        for b in range(_NBUF):
            start(b, b)

        def group_body(g, carry):
            for b in range(_NBUF):
                c = g * _NBUF + b
                wait(c, b)
                reduce_chunk(c, b)
                start(c + _NBUF, b)
            return carry

        lax.fori_loop(0, n_groups - 1, group_body, 0)

        for b in range(_NBUF):
            c = (n_groups - 1) * _NBUF + b
            wait(c, b)
            reduce_chunk(c, b)

        pltpu.sync_copy(out_v, out_hbm.at[pl.ds(wid * bags_per_w, bags_per_w)])

    return k


def kernel(input_, weight):
    B, H = input_.shape
    V = weight.shape[0]
    idx2 = input_.reshape(B // _BAGS_PER_CHUNK, _IDX_PER_CHUNK)
    return _make_kernel(B, V)(idx2, weight)
